# baseline (speedup 1.0000x reference)
import functools

import jax
import jax.numpy as jnp
from jax import lax
from jax.experimental import pallas as pl
from jax.experimental.pallas import tpu as pltpu
from jax.experimental.pallas import tpu_sc as plsc

_NC = 2
_NS = 16
_NW = _NC * _NS
_C = 128
_B = 8


def _sc_degree(Npad, K):
    TR = Npad // _NS
    mesh = plsc.VectorSubcoreMesh(
        core_axis_name="c", subcore_axis_name="s",
        num_cores=_NC, num_subcores=_NS)

    @functools.partial(
        pl.kernel,
        out_type=jax.ShapeDtypeStruct((_NC, Npad), jnp.float32),
        mesh=mesh,
        scratch_types=[
            pltpu.VMEM((K, _C), jnp.int32),
            pltpu.VMEM((_C,), jnp.float32),
            pltpu.VMEM((TR,), jnp.float32),
            pltpu.VMEM_SHARED((Npad,), jnp.float32),
            pltpu.SemaphoreType.DMA,
        ],
    )
    def deg_kernel(dst_hbm, out_hbm, dst_v, ones_v, z_v, acc, sem):
        cid = lax.axis_index("c")
        sid = lax.axis_index("s")
        wid = cid * _NS + sid

        def fill_zero(i, carry):
            z_v[pl.ds(i * 16, 16)] = jnp.zeros((16,), jnp.float32)
            return carry

        lax.fori_loop(0, TR // 16, fill_zero, 0)
        for i in range(_C // 16):
            ones_v[pl.ds(i * 16, 16)] = jnp.ones((16,), jnp.float32)
        pltpu.sync_copy(z_v, acc.at[pl.ds(sid * TR, TR)])
        pltpu.sync_copy(dst_hbm.at[wid], dst_v)
        plsc.subcore_barrier()

        def body(j, carry):
            pltpu.async_copy(ones_v, acc.at[dst_v.at[j]], sem, add=True)
            return carry

        lax.fori_loop(0, K, body, 0)

        def drain(j, carry):
            pltpu.make_async_copy(ones_v, acc.at[dst_v.at[0]], sem).wait()
            return carry

        lax.fori_loop(0, K, drain, 0)
        plsc.subcore_barrier()
        pltpu.sync_copy(acc.at[pl.ds(sid * TR, TR)],
                        out_hbm.at[cid, pl.ds(sid * TR, TR)])

    return deg_kernel


def _sc_propagate(Npad, K, D):
    TR = Npad // _NS
    mesh = plsc.VectorSubcoreMesh(
        core_axis_name="c", subcore_axis_name="s",
        num_cores=_NC, num_subcores=_NS)

    assert K % _B == 0
    NB = K // _B

    @functools.partial(
        pl.kernel,
        out_type=jax.ShapeDtypeStruct((_NC, Npad, D), jnp.float32),
        mesh=mesh,
        scratch_types=[
            pltpu.VMEM((2, _B, _C), jnp.int32),
            pltpu.VMEM((2, _B, _C), jnp.int32),
            pltpu.VMEM((_C, D), jnp.float32),
            pltpu.VMEM((_C, D), jnp.float32),
            pltpu.VMEM_SHARED((Npad, D), jnp.float32),
            pltpu.SemaphoreType.DMA,
            pltpu.SemaphoreType.DMA,
            pltpu.SemaphoreType.DMA,
            pltpu.SemaphoreType.DMA,
            pltpu.SemaphoreType.DMA,
        ],
    )
    def prop_kernel(y_hbm, src_hbm, dst_hbm, out_hbm,
                    src_v, dst_v, rows_a, rows_b, acc,
                    sem_i, sem_g0, sem_g1, sem_s0, sem_s1):
        cid = lax.axis_index("c")
        sid = lax.axis_index("s")
        wid = cid * _NS + sid
        rows = (rows_a, rows_b)
        sem_g = (sem_g0, sem_g1)
        sem_s = (sem_s0, sem_s1)

        def zero_row(r, carry):
            for i in range(D // 16):
                rows_a[r, pl.ds(i * 16, 16)] = jnp.zeros((16,), jnp.float32)
            return carry

        lax.fori_loop(0, _C, zero_row, 0)
        for t in range(TR // _C):
            pltpu.sync_copy(rows_a, acc.at[pl.ds(sid * TR + t * _C, _C)])
        pltpu.sync_copy(src_hbm.at[wid, pl.ds(0, _B)], src_v.at[0])
        pltpu.sync_copy(dst_hbm.at[wid, pl.ds(0, _B)], dst_v.at[0])
        plsc.subcore_barrier()

        pltpu.async_copy(y_hbm.at[src_v.at[0, 0]], rows_a, sem_g0)

        def outer(o, carry):
            s = o % 2

            @pl.when(o + 1 < NB)
            def _():
                nxt = (o + 1) * _B
                pltpu.async_copy(src_hbm.at[wid, pl.ds(nxt, _B)],
                                 src_v.at[1 - s], sem_i)
                pltpu.async_copy(dst_hbm.at[wid, pl.ds(nxt, _B)],
                                 dst_v.at[1 - s], sem_i)

            for b in range(_B):
                r = b % 2
                pltpu.make_async_copy(y_hbm.at[src_v.at[s, b]], rows[r],
                                      sem_g[r]).wait()
                pltpu.async_copy(rows[r], acc.at[dst_v.at[s, b]], sem_s[r],
                                 add=True)
                if b == 0:
                    @pl.when(o > 0)
                    def _():
                        pltpu.make_async_copy(rows[1], acc.at[dst_v.at[s, 0]],
                                              sem_s[1]).wait()
                else:
                    pltpu.make_async_copy(rows[1 - r], acc.at[dst_v.at[s, b]],
                                          sem_s[1 - r]).wait()
                if b < _B - 1:
                    pltpu.async_copy(y_hbm.at[src_v.at[s, b + 1]], rows[1 - r],
                                     sem_g[1 - r])
                else:
                    @pl.when(o + 1 < NB)
                    def _():
                        pltpu.make_async_copy(
                            src_hbm.at[wid, pl.ds((o + 1) * _B, _B)],
                            src_v.at[1 - s], sem_i).wait()
                        pltpu.make_async_copy(
                            dst_hbm.at[wid, pl.ds((o + 1) * _B, _B)],
                            dst_v.at[1 - s], sem_i).wait()
                        pltpu.async_copy(y_hbm.at[src_v.at[1 - s, 0]],
                                         rows[1 - r], sem_g[1 - r])
            return carry

        lax.fori_loop(0, NB, outer, 0)
        pltpu.make_async_copy(rows[1], acc.at[dst_v.at[(NB - 1) % 2, _B - 1]],
                              sem_s[1]).wait()
        plsc.subcore_barrier()
        for t in range(TR // _C):
            pltpu.sync_copy(acc.at[pl.ds(sid * TR + t * _C, _C)],
                            out_hbm.at[cid, pl.ds(sid * TR + t * _C, _C)])

    return prop_kernel


def _dinv_of(dg_block):
    return lax.rsqrt(dg_block[:, 0:1] + dg_block[:, 1:2] + 1.0)


def _tc_scale_matmul(x_pad, W1, degt, BR):
    Npad, Din = x_pad.shape
    Dh = W1.shape[1]

    def body(x_ref, w_ref, dg_ref, y_ref):
        dinv = _dinv_of(dg_ref[...])
        y_ref[...] = dinv * jnp.dot(x_ref[...], w_ref[...],
                                    preferred_element_type=jnp.float32)

    return pl.pallas_call(
        body,
        grid=(Npad // BR,),
        in_specs=[
            pl.BlockSpec((BR, Din), lambda i: (i, 0)),
            pl.BlockSpec((Din, Dh), lambda i: (0, 0)),
            pl.BlockSpec((BR, 2), lambda i: (i, 0)),
        ],
        out_specs=pl.BlockSpec((BR, Dh), lambda i: (i, 0)),
        out_shape=jax.ShapeDtypeStruct((Npad, Dh), jnp.float32),
    )(x_pad, W1, degt)


def _tc_layer1_post(Zp, Y1, degt, b1, BR):
    Npad, Dh = Y1.shape

    def body(zp_ref, y_ref, dg_ref, b_ref, out_ref):
        dinv = _dinv_of(dg_ref[...])
        z = zp_ref[0] + zp_ref[1] + y_ref[...]
        h = jnp.maximum(dinv * z + b_ref[...], 0.0)
        out_ref[...] = dinv * h

    return pl.pallas_call(
        body,
        grid=(Npad // BR,),
        in_specs=[
            pl.BlockSpec((2, BR, Dh), lambda i: (0, i, 0)),
            pl.BlockSpec((BR, Dh), lambda i: (i, 0)),
            pl.BlockSpec((BR, 2), lambda i: (i, 0)),
            pl.BlockSpec((1, Dh), lambda i: (0, 0)),
        ],
        out_specs=pl.BlockSpec((BR, Dh), lambda i: (i, 0)),
        out_shape=jax.ShapeDtypeStruct((Npad, Dh), jnp.float32),
    )(Zp, Y1, degt, b1)


def _tc_heads(Zp2, Y2, degt, W_mu, b_mu, W_ls, b_ls, BR):
    Npad, Dh = Y2.shape
    Dz = W_mu.shape[1]

    def body(zp_ref, y_ref, dg_ref, wmu_ref, bmu_ref, wls_ref, bls_ref,
             mu_ref, ls_ref):
        dinv = _dinv_of(dg_ref[...])
        p = dinv * (zp_ref[0] + zp_ref[1] + y_ref[...])
        mu_ref[...] = jnp.dot(p, wmu_ref[...],
                              preferred_element_type=jnp.float32) + bmu_ref[...]
        ls_ref[...] = jnp.dot(p, wls_ref[...],
                              preferred_element_type=jnp.float32) + bls_ref[...]

    return pl.pallas_call(
        body,
        grid=(Npad // BR,),
        in_specs=[
            pl.BlockSpec((2, BR, Dh), lambda i: (0, i, 0)),
            pl.BlockSpec((BR, Dh), lambda i: (i, 0)),
            pl.BlockSpec((BR, 2), lambda i: (i, 0)),
            pl.BlockSpec((Dh, Dz), lambda i: (0, 0)),
            pl.BlockSpec((1, Dz), lambda i: (0, 0)),
            pl.BlockSpec((Dh, Dz), lambda i: (0, 0)),
            pl.BlockSpec((1, Dz), lambda i: (0, 0)),
        ],
        out_specs=[
            pl.BlockSpec((BR, Dz), lambda i: (i, 0)),
            pl.BlockSpec((BR, Dz), lambda i: (i, 0)),
        ],
        out_shape=[
            jax.ShapeDtypeStruct((Npad, Dz), jnp.float32),
            jax.ShapeDtypeStruct((Npad, Dz), jnp.float32),
        ],
    )(Zp2, Y2, degt, W_mu, b_mu, W_ls, b_ls)


def kernel(x, edge_index, W1, b1, W_mu, b_mu, W_ls, b_ls):
    N, Din = x.shape
    Dh = W1.shape[1]
    Dz = W_mu.shape[1]
    E = edge_index.shape[1]

    K = -(-E // (_NW * _C * _B)) * _B
    Epad = _NW * K * _C
    Npad = -(-(N + 1) // (_NS * _C)) * (_NS * _C)
    BR = 1280 if Npad % 1280 == 0 else Npad // _NS

    src = edge_index[0].astype(jnp.int32)
    dst = edge_index[1].astype(jnp.int32)
    if Epad != E:
        fill = jnp.full((Epad - E,), N, dtype=jnp.int32)
        src = jnp.concatenate([src, fill])
        dst = jnp.concatenate([dst, fill])
    src3 = src.reshape(_NW, K, _C)
    dst3 = dst.reshape(_NW, K, _C)

    x_pad = jnp.zeros((Npad, Din), jnp.float32).at[:N].set(x)

    degp = _sc_degree(Npad, K)(dst3)
    degt = jnp.transpose(degp)

    Y1 = _tc_scale_matmul(x_pad, W1, degt, BR)
    Zp1 = _sc_propagate(Npad, K, Dh)(Y1, src3, dst3)
    Y2 = _tc_layer1_post(Zp1, Y1, degt, b1.reshape(1, Dh), BR)
    Zp2 = _sc_propagate(Npad, K, Dh)(Y2, src3, dst3)
    mu_p, ls_p = _tc_heads(Zp2, Y2, degt, W_mu, b_mu.reshape(1, Dz),
                           W_ls, b_ls.reshape(1, Dz), BR)
    return mu_p[:N], ls_p[:N]


# speedup vs baseline: 1.0406x; 1.0406x over previous
import functools

import jax
import jax.numpy as jnp
from jax import lax
from jax.experimental import pallas as pl
from jax.experimental.pallas import tpu as pltpu
from jax.experimental.pallas import tpu_sc as plsc

_NC = 2
_NS = 16
_C = 128
_B = 8
_FRAC = 0.5


def _chunk_split(E):
    KT = -(-E // (_NS * _C))
    KT = -(-KT // (2 * _B)) * (2 * _B)
    k0 = int(round(KT * _FRAC / _B)) * _B
    k0 = min(max(k0, _B), KT - _B)
    k1 = KT - k0
    TK = _NS * KT
    TKa = TK + (k0 - k1)
    return k0, k1, TK, TKa


def _mesh():
    return plsc.VectorSubcoreMesh(
        core_axis_name="c", subcore_axis_name="s",
        num_cores=_NC, num_subcores=_NS)


def _tile_range(cid, sid, k0, k1):
    ksel = jnp.where(cid == 0, k0, k1)
    start = cid * (_NS * k0) + sid * ksel
    return ksel, start


def _sc_degree(Npad, k0, k1, TKa):
    TR = Npad // _NS

    @functools.partial(
        pl.kernel,
        out_type=jax.ShapeDtypeStruct((_NC, Npad), jnp.float32),
        mesh=_mesh(),
        scratch_types=[
            pltpu.VMEM((k0, _C), jnp.int32),
            pltpu.VMEM((_C,), jnp.float32),
            pltpu.VMEM((TR,), jnp.float32),
            pltpu.VMEM_SHARED((Npad,), jnp.float32),
            pltpu.SemaphoreType.DMA,
        ],
    )
    def deg_kernel(dst_hbm, out_hbm, dst_v, ones_v, z_v, acc, sem):
        cid = lax.axis_index("c")
        sid = lax.axis_index("s")
        ksel, start = _tile_range(cid, sid, k0, k1)

        def fill_zero(i, carry):
            z_v[pl.ds(i * 16, 16)] = jnp.zeros((16,), jnp.float32)
            return carry

        lax.fori_loop(0, TR // 16, fill_zero, 0)
        for i in range(_C // 16):
            ones_v[pl.ds(i * 16, 16)] = jnp.ones((16,), jnp.float32)
        pltpu.sync_copy(z_v, acc.at[pl.ds(sid * TR, TR)])
        pltpu.sync_copy(dst_hbm.at[pl.ds(start, k0)], dst_v)
        plsc.subcore_barrier()

        def body(j, carry):
            pltpu.async_copy(ones_v, acc.at[dst_v.at[j]], sem, add=True)
            return carry

        lax.fori_loop(0, ksel, body, 0)

        def drain(j, carry):
            pltpu.make_async_copy(ones_v, acc.at[dst_v.at[0]], sem).wait()
            return carry

        lax.fori_loop(0, ksel, drain, 0)
        plsc.subcore_barrier()
        pltpu.sync_copy(acc.at[pl.ds(sid * TR, TR)],
                        out_hbm.at[cid, pl.ds(sid * TR, TR)])

    return deg_kernel


def _sc_propagate(Npad, k0, k1, TKa, D):
    TR = Npad // _NS
    assert k0 % (2 * _B) == 0 and k1 % (2 * _B) == 0

    nb0, nb1 = k0 // _B, k1 // _B

    @functools.partial(
        pl.kernel,
        out_type=jax.ShapeDtypeStruct((_NC, Npad, D), jnp.float32),
        mesh=_mesh(),
        scratch_types=[
            pltpu.VMEM((2, _B, _C), jnp.int32),
            pltpu.VMEM((2, _B, _C), jnp.int32),
            pltpu.VMEM((_C, D), jnp.float32),
            pltpu.VMEM((_C, D), jnp.float32),
            pltpu.VMEM_SHARED((Npad, D), jnp.float32),
            pltpu.SemaphoreType.DMA,
            pltpu.SemaphoreType.DMA,
            pltpu.SemaphoreType.DMA,
        ],
    )
    def prop_kernel(y_hbm, src_hbm, dst_hbm, out_hbm,
                    src_v, dst_v, rows_a, rows_b, acc, sem_i, sem_g0, sem_g1):
        cid = lax.axis_index("c")
        sid = lax.axis_index("s")
        ksel, start = _tile_range(cid, sid, k0, k1)
        nb = ksel // _B
        rows = (rows_a, rows_b)
        sem_g = (sem_g0, sem_g1)

        def zero_row(r, carry):
            for i in range(D // 16):
                rows_a[r, pl.ds(i * 16, 16)] = jnp.zeros((16,), jnp.float32)
            return carry

        lax.fori_loop(0, _C, zero_row, 0)
        for t in range(TR // _C):
            pltpu.sync_copy(rows_a, acc.at[pl.ds(sid * TR + t * _C, _C)])
        pltpu.sync_copy(src_hbm.at[pl.ds(start, _B)], src_v.at[0])
        pltpu.sync_copy(dst_hbm.at[pl.ds(start, _B)], dst_v.at[0])
        plsc.subcore_barrier()

        pltpu.async_copy(y_hbm.at[src_v.at[0, 0]], rows_a, sem_g0)

        def outer(o, carry):
            s = o % 2
            blk = start + o * _B

            @pl.when(o + 1 < nb)
            def _():
                pltpu.async_copy(src_hbm.at[pl.ds(blk + _B, _B)],
                                 src_v.at[1 - s], sem_i)
                pltpu.async_copy(dst_hbm.at[pl.ds(blk + _B, _B)],
                                 dst_v.at[1 - s], sem_i)

            for b in range(_B):
                r = b % 2
                pltpu.make_async_copy(y_hbm.at[src_v.at[s, b]], rows[r],
                                      sem_g[r]).wait()
                if b < _B - 1:
                    pltpu.async_copy(y_hbm.at[src_v.at[s, b + 1]],
                                     rows[1 - r], sem_g[1 - r])
                else:
                    @pl.when(o + 1 < nb)
                    def _():
                        pltpu.make_async_copy(
                            src_hbm.at[pl.ds(blk + _B, _B)],
                            src_v.at[1 - s], sem_i).wait()
                        pltpu.make_async_copy(
                            dst_hbm.at[pl.ds(blk + _B, _B)],
                            dst_v.at[1 - s], sem_i).wait()
                        pltpu.async_copy(y_hbm.at[src_v.at[1 - s, 0]],
                                         rows[1 - r], sem_g[1 - r])
                pltpu.sync_copy(rows[r], acc.at[dst_v.at[s, b]], add=True)
            return carry

        lax.fori_loop(0, nb, outer, 0)
        plsc.subcore_barrier()
        for t in range(TR // _C):
            pltpu.sync_copy(acc.at[pl.ds(sid * TR + t * _C, _C)],
                            out_hbm.at[cid, pl.ds(sid * TR + t * _C, _C)])

    return prop_kernel


def _dinv_of(dg_block):
    return lax.rsqrt(dg_block[:, 0:1] + dg_block[:, 1:2] + 1.0)


def _tc_scale_matmul(x, W1, degt, BR):
    N, Din = x.shape
    Dh = W1.shape[1]

    def body(x_ref, w_ref, dg_ref, y_ref):
        dinv = _dinv_of(dg_ref[...])
        y_ref[...] = dinv * jnp.dot(x_ref[...], w_ref[...],
                                    preferred_element_type=jnp.float32)

    return pl.pallas_call(
        body,
        grid=(N // BR,),
        in_specs=[
            pl.BlockSpec((BR, Din), lambda i: (i, 0)),
            pl.BlockSpec((Din, Dh), lambda i: (0, 0)),
            pl.BlockSpec((BR, 2), lambda i: (i, 0)),
        ],
        out_specs=pl.BlockSpec((BR, Dh), lambda i: (i, 0)),
        out_shape=jax.ShapeDtypeStruct((N, Dh), jnp.float32),
    )(x, W1, degt)


def _tc_layer1_post(Zp, Y1, degt, b1, BR):
    N, Dh = Y1.shape

    def body(zp_ref, y_ref, dg_ref, b_ref, out_ref):
        dinv = _dinv_of(dg_ref[...])
        z = zp_ref[0] + zp_ref[1] + y_ref[...]
        h = jnp.maximum(dinv * z + b_ref[...], 0.0)
        out_ref[...] = dinv * h

    return pl.pallas_call(
        body,
        grid=(N // BR,),
        in_specs=[
            pl.BlockSpec((2, BR, Dh), lambda i: (0, i, 0)),
            pl.BlockSpec((BR, Dh), lambda i: (i, 0)),
            pl.BlockSpec((BR, 2), lambda i: (i, 0)),
            pl.BlockSpec((1, Dh), lambda i: (0, 0)),
        ],
        out_specs=pl.BlockSpec((BR, Dh), lambda i: (i, 0)),
        out_shape=jax.ShapeDtypeStruct((N, Dh), jnp.float32),
    )(Zp, Y1, degt, b1)


def _tc_heads(Zp2, Y2, degt, W_mu, b_mu, W_ls, b_ls, BR):
    N, Dh = Y2.shape
    Dz = W_mu.shape[1]

    def body(zp_ref, y_ref, dg_ref, wmu_ref, bmu_ref, wls_ref, bls_ref,
             mu_ref, ls_ref):
        dinv = _dinv_of(dg_ref[...])
        p = dinv * (zp_ref[0] + zp_ref[1] + y_ref[...])
        mu_ref[...] = jnp.dot(p, wmu_ref[...],
                              preferred_element_type=jnp.float32) + bmu_ref[...]
        ls_ref[...] = jnp.dot(p, wls_ref[...],
                              preferred_element_type=jnp.float32) + bls_ref[...]

    return pl.pallas_call(
        body,
        grid=(N // BR,),
        in_specs=[
            pl.BlockSpec((2, BR, Dh), lambda i: (0, i, 0)),
            pl.BlockSpec((BR, Dh), lambda i: (i, 0)),
            pl.BlockSpec((BR, 2), lambda i: (i, 0)),
            pl.BlockSpec((Dh, Dz), lambda i: (0, 0)),
            pl.BlockSpec((1, Dz), lambda i: (0, 0)),
            pl.BlockSpec((Dh, Dz), lambda i: (0, 0)),
            pl.BlockSpec((1, Dz), lambda i: (0, 0)),
        ],
        out_specs=[
            pl.BlockSpec((BR, Dz), lambda i: (i, 0)),
            pl.BlockSpec((BR, Dz), lambda i: (i, 0)),
        ],
        out_shape=[
            jax.ShapeDtypeStruct((N, Dz), jnp.float32),
            jax.ShapeDtypeStruct((N, Dz), jnp.float32),
        ],
    )(Zp2, Y2, degt, W_mu, b_mu, W_ls, b_ls)


def kernel(x, edge_index, W1, b1, W_mu, b_mu, W_ls, b_ls):
    N, Din = x.shape
    Dh = W1.shape[1]
    Dz = W_mu.shape[1]
    E = edge_index.shape[1]

    k0, k1, TK, TKa = _chunk_split(E)
    Npad = -(-(N + 1) // (_NS * _C)) * (_NS * _C)
    BR = 1000 if N % 1000 == 0 else N // 8

    src = edge_index[0].astype(jnp.int32)
    dst = edge_index[1].astype(jnp.int32)
    src3 = jnp.concatenate(
        [src, jnp.zeros((TKa * _C - E,), jnp.int32)]).reshape(TKa, _C)
    dst3 = jnp.concatenate(
        [dst, jnp.full((TKa * _C - E,), N, jnp.int32)]).reshape(TKa, _C)

    degp = _sc_degree(Npad, k0, k1, TKa)(dst3)
    degt = jnp.transpose(degp)

    Y1 = _tc_scale_matmul(x, W1, degt, BR)
    Zp1 = _sc_propagate(Npad, k0, k1, TKa, Dh)(Y1, src3, dst3)
    Y2 = _tc_layer1_post(Zp1, Y1, degt, b1.reshape(1, Dh), BR)
    Zp2 = _sc_propagate(Npad, k0, k1, TKa, Dh)(Y2, src3, dst3)
    mu, ls = _tc_heads(Zp2, Y2, degt, W_mu, b_mu.reshape(1, Dz),
                       W_ls, b_ls.reshape(1, Dz), BR)
    return mu, ls


# speedup vs baseline: 1.3799x; 1.3260x over previous
import functools

import jax
import jax.numpy as jnp
from jax import lax
from jax.experimental import pallas as pl
from jax.experimental.pallas import tpu as pltpu
from jax.experimental.pallas import tpu_sc as plsc

_NC = 2
_NS = 16
_NW = _NC * _NS
_C = 128


def _mesh():
    return plsc.VectorSubcoreMesh(
        core_axis_name="c", subcore_axis_name="s",
        num_cores=_NC, num_subcores=_NS)


def _sc_degree(Npad, K):
    TR = Npad // _NS

    @functools.partial(
        pl.kernel,
        out_type=jax.ShapeDtypeStruct((_NC, Npad), jnp.float32),
        mesh=_mesh(),
        scratch_types=[
            pltpu.VMEM((K, _C), jnp.int32),
            pltpu.VMEM((_C,), jnp.float32),
            pltpu.VMEM((TR,), jnp.float32),
            pltpu.VMEM_SHARED((Npad,), jnp.float32),
            pltpu.SemaphoreType.DMA,
        ],
    )
    def deg_kernel(dst_hbm, out_hbm, dst_v, ones_v, z_v, acc, sem):
        cid = lax.axis_index("c")
        sid = lax.axis_index("s")
        wid = cid * _NS + sid

        def fill_zero(i, carry):
            z_v[pl.ds(i * 16, 16)] = jnp.zeros((16,), jnp.float32)
            return carry

        lax.fori_loop(0, TR // 16, fill_zero, 0)
        for i in range(_C // 16):
            ones_v[pl.ds(i * 16, 16)] = jnp.ones((16,), jnp.float32)
        pltpu.sync_copy(z_v, acc.at[pl.ds(sid * TR, TR)])
        pltpu.sync_copy(dst_hbm.at[wid], dst_v)
        plsc.subcore_barrier()

        def body(j, carry):
            pltpu.async_copy(ones_v, acc.at[dst_v.at[j]], sem, add=True)
            return carry

        lax.fori_loop(0, K, body, 0)

        def drain(j, carry):
            pltpu.make_async_copy(ones_v, acc.at[dst_v.at[0]], sem).wait()
            return carry

        lax.fori_loop(0, K, drain, 0)
        plsc.subcore_barrier()
        pltpu.sync_copy(acc.at[pl.ds(sid * TR, TR)],
                        out_hbm.at[cid, pl.ds(sid * TR, TR)])

    return deg_kernel


def _sc_propagate(Npad, K, D):
    TR = Npad // _NS

    @functools.partial(
        pl.kernel,
        out_type=jax.ShapeDtypeStruct((_NC, Npad, D), jnp.float32),
        mesh=_mesh(),
        scratch_types=[
            pltpu.VMEM((K, _C), jnp.int32),
            pltpu.VMEM((K, _C), jnp.int32),
            pltpu.VMEM((_C, D), jnp.float32),
            pltpu.VMEM_SHARED((Npad, D), jnp.float32),
            pltpu.SemaphoreType.DMA,
        ],
    )
    def prop_kernel(y_hbm, src_hbm, dst_hbm, out_hbm,
                    src_v, dst_v, rows_v, acc, sem):
        cid = lax.axis_index("c")
        sid = lax.axis_index("s")
        wid = cid * _NS + sid

        def zero_row(r, carry):
            for i in range(D // 16):
                rows_v[r, pl.ds(i * 16, 16)] = jnp.zeros((16,), jnp.float32)
            return carry

        lax.fori_loop(0, _C, zero_row, 0)
        for t in range(TR // _C):
            pltpu.sync_copy(rows_v, acc.at[pl.ds(sid * TR + t * _C, _C)])
        pltpu.sync_copy(src_hbm.at[wid], src_v)
        pltpu.sync_copy(dst_hbm.at[wid], dst_v)
        plsc.subcore_barrier()

        def body(j, carry):
            pltpu.async_copy(y_hbm.at[src_v.at[j]], rows_v, sem).wait()
            pltpu.sync_copy(rows_v, acc.at[dst_v.at[j]], add=True)
            return carry

        lax.fori_loop(0, K, body, 0)
        plsc.subcore_barrier()
        for t in range(TR // _C):
            pltpu.sync_copy(acc.at[pl.ds(sid * TR + t * _C, _C)],
                            out_hbm.at[cid, pl.ds(sid * TR + t * _C, _C)])

    return prop_kernel


def _dinv_of(dg_block):
    return lax.rsqrt(dg_block[:, 0:1] + dg_block[:, 1:2] + 1.0)


def _tc_scale_matmul(x, W1, degt, BR):
    N, Din = x.shape
    Dh = W1.shape[1]

    def body(x_ref, w_ref, dg_ref, y_ref):
        dinv = _dinv_of(dg_ref[...])
        y_ref[...] = dinv * jnp.dot(x_ref[...], w_ref[...],
                                    preferred_element_type=jnp.float32)

    return pl.pallas_call(
        body,
        grid=(N // BR,),
        in_specs=[
            pl.BlockSpec((BR, Din), lambda i: (i, 0)),
            pl.BlockSpec((Din, Dh), lambda i: (0, 0)),
            pl.BlockSpec((BR, 2), lambda i: (i, 0)),
        ],
        out_specs=pl.BlockSpec((BR, Dh), lambda i: (i, 0)),
        out_shape=jax.ShapeDtypeStruct((N, Dh), jnp.float32),
    )(x, W1, degt)


def _tc_layer1_post(Zp, Y1, degt, b1, BR):
    N, Dh = Y1.shape

    def body(zp_ref, y_ref, dg_ref, b_ref, out_ref):
        dinv = _dinv_of(dg_ref[...])
        z = zp_ref[0] + zp_ref[1] + y_ref[...]
        h = jnp.maximum(dinv * z + b_ref[...], 0.0)
        out_ref[...] = dinv * h

    return pl.pallas_call(
        body,
        grid=(N // BR,),
        in_specs=[
            pl.BlockSpec((2, BR, Dh), lambda i: (0, i, 0)),
            pl.BlockSpec((BR, Dh), lambda i: (i, 0)),
            pl.BlockSpec((BR, 2), lambda i: (i, 0)),
            pl.BlockSpec((1, Dh), lambda i: (0, 0)),
        ],
        out_specs=pl.BlockSpec((BR, Dh), lambda i: (i, 0)),
        out_shape=jax.ShapeDtypeStruct((N, Dh), jnp.float32),
    )(Zp, Y1, degt, b1)


def _tc_heads(Zp2, Y2, degt, W_mu, b_mu, W_ls, b_ls, BR):
    N, Dh = Y2.shape
    Dz = W_mu.shape[1]

    def body(zp_ref, y_ref, dg_ref, wmu_ref, bmu_ref, wls_ref, bls_ref,
             mu_ref, ls_ref):
        dinv = _dinv_of(dg_ref[...])
        p = dinv * (zp_ref[0] + zp_ref[1] + y_ref[...])
        mu_ref[...] = jnp.dot(p, wmu_ref[...],
                              preferred_element_type=jnp.float32) + bmu_ref[...]
        ls_ref[...] = jnp.dot(p, wls_ref[...],
                              preferred_element_type=jnp.float32) + bls_ref[...]

    return pl.pallas_call(
        body,
        grid=(N // BR,),
        in_specs=[
            pl.BlockSpec((2, BR, Dh), lambda i: (0, i, 0)),
            pl.BlockSpec((BR, Dh), lambda i: (i, 0)),
            pl.BlockSpec((BR, 2), lambda i: (i, 0)),
            pl.BlockSpec((Dh, Dz), lambda i: (0, 0)),
            pl.BlockSpec((1, Dz), lambda i: (0, 0)),
            pl.BlockSpec((Dh, Dz), lambda i: (0, 0)),
            pl.BlockSpec((1, Dz), lambda i: (0, 0)),
        ],
        out_specs=[
            pl.BlockSpec((BR, Dz), lambda i: (i, 0)),
            pl.BlockSpec((BR, Dz), lambda i: (i, 0)),
        ],
        out_shape=[
            jax.ShapeDtypeStruct((N, Dz), jnp.float32),
            jax.ShapeDtypeStruct((N, Dz), jnp.float32),
        ],
    )(Zp2, Y2, degt, W_mu, b_mu, W_ls, b_ls)


def kernel(x, edge_index, W1, b1, W_mu, b_mu, W_ls, b_ls):
    N, Din = x.shape
    Dh = W1.shape[1]
    Dz = W_mu.shape[1]
    E = edge_index.shape[1]

    K = -(-E // (_NW * _C))
    Epad = _NW * K * _C
    Npad = -(-(N + 1) // (_NS * _C)) * (_NS * _C)
    BR = 1000 if N % 1000 == 0 else N // 8

    src = edge_index[0].astype(jnp.int32)
    dst = edge_index[1].astype(jnp.int32)
    if Epad != E:
        src = jnp.concatenate([src, jnp.zeros((Epad - E,), jnp.int32)])
        dst = jnp.concatenate([dst, jnp.full((Epad - E,), N, jnp.int32)])
    src3 = src.reshape(_NW, K, _C)
    dst3 = dst.reshape(_NW, K, _C)

    degp = _sc_degree(Npad, K)(dst3)
    degt = jnp.transpose(degp)

    Y1 = _tc_scale_matmul(x, W1, degt, BR)
    Zp1 = _sc_propagate(Npad, K, Dh)(Y1, src3, dst3)
    Y2 = _tc_layer1_post(Zp1, Y1, degt, b1.reshape(1, Dh), BR)
    Zp2 = _sc_propagate(Npad, K, Dh)(Y2, src3, dst3)
    mu, ls = _tc_heads(Zp2, Y2, degt, W_mu, b_mu.reshape(1, Dz),
                       W_ls, b_ls.reshape(1, Dz), BR)
    return mu, ls
